# Initial kernel scaffold; baseline (speedup 1.0000x reference)
import jax
import jax.numpy as jnp
from jax.experimental import pallas as pl


def kernel(x, edge_index, W1, b1, W2, b2, g1, bt1, g2, bt2, g3, bt3, Wm, bm):
    raise NotImplementedError("write your pallas kernel here")


# speedup vs baseline: 9.4944x; 9.4944x over previous
import jax
import jax.numpy as jnp
from jax import lax
from jax.experimental import pallas as pl
from jax.experimental.pallas import tpu as pltpu
from jax.experimental.pallas import tpu_sc as plsc

N = 10000
D = 128
DOUT = 40
NPAD = 10240
CH = 128
NC = 2
NS = 16
NW = NC * NS
RPT = NPAD // NS

_MESH = plsc.VectorSubcoreMesh(
    core_axis_name="c", subcore_axis_name="s", num_cores=NC, num_subcores=NS)


def _deg_body(dstm, ones, zeros, out, didx_v, ones_v, acc_sh):
    c = lax.axis_index("c")
    s = lax.axis_index("s")
    w = s * NC + c
    perw = dstm.shape[0] // NW
    rows = pl.ds(s * RPT, RPT)
    pltpu.sync_copy(zeros.at[rows], acc_sh.at[rows])
    pltpu.sync_copy(ones, ones_v)
    pltpu.sync_copy(dstm.at[pl.ds(w * perw, perw)], didx_v)
    plsc.subcore_barrier()

    def step(j, carry):
        pltpu.sync_copy(ones_v, acc_sh.at[didx_v.at[j]], add=True)
        return carry

    lax.fori_loop(0, perw, step, 0)
    plsc.subcore_barrier()
    pltpu.sync_copy(acc_sh.at[rows], out.at[c, rows, :])


def _agg_body(srcm, dstm, hs, zeros, out, sidx_v, didx_v, rows_v, acc_sh, gsem):
    c = lax.axis_index("c")
    s = lax.axis_index("s")
    w = s * NC + c
    perw = srcm.shape[0] // NW
    rows = pl.ds(s * RPT, RPT)
    pltpu.sync_copy(zeros.at[rows], acc_sh.at[rows])
    pltpu.sync_copy(srcm.at[pl.ds(w * perw, perw)], sidx_v)
    pltpu.sync_copy(dstm.at[pl.ds(w * perw, perw)], didx_v)
    plsc.subcore_barrier()

    def step(j, carry):
        pltpu.async_copy(hs.at[sidx_v.at[j]], rows_v, gsem).wait()
        pltpu.sync_copy(rows_v, acc_sh.at[didx_v.at[j]], add=True)
        return carry

    lax.fori_loop(0, perw, step, 0)
    plsc.subcore_barrier()
    pltpu.sync_copy(acc_sh.at[rows], out.at[c, rows, :])


def _tc1_body(x_ref, w1_ref, degp_ref, hs_ref, dinv_ref):
    deg = degp_ref[0] + degp_ref[1] + 1.0
    dinv = lax.rsqrt(deg)
    dinv_ref[...] = dinv
    h = jnp.dot(x_ref[...], w1_ref[...], preferred_element_type=jnp.float32)
    hs_ref[...] = h * dinv[:N, 0:1]


def _bn_relu(conv, g, bt):
    mu = jnp.mean(conv, axis=0, keepdims=True)
    xc = conv - mu
    var = jnp.mean(xc * xc, axis=0, keepdims=True)
    return jnp.maximum(xc * lax.rsqrt(var + 1e-5) * g + bt, 0.0)


def _tc2_body(aggp_ref, hs_ref, dinvp_ref, b1_ref, g1_ref, bt1_ref, w2_ref,
              hs2_ref):
    dcol = dinvp_ref[:, 0:1][:N]
    conv = (aggp_ref[0, :N, :] + aggp_ref[1, :N, :] + hs_ref[...]) * dcol
    h = _bn_relu(conv + b1_ref[...], g1_ref[...], bt1_ref[...])
    h2 = jnp.dot(h, w2_ref[...], preferred_element_type=jnp.float32)
    hs2_ref[...] = h2 * dcol


def _tc3_body(aggp_ref, hs2_ref, dinvp_ref, b2_ref, g2_ref, bt2_ref,
              g3_ref, bt3_ref, wm_ref, bm_ref, out_ref):
    dcol = dinvp_ref[:, 0:1][:N]
    conv = (aggp_ref[0, :N, :] + aggp_ref[1, :N, :] + hs2_ref[...]) * dcol
    h = _bn_relu(conv + b2_ref[...], g2_ref[...], bt2_ref[...])
    mu = jnp.mean(h, axis=0, keepdims=True)
    xc = h - mu
    var = jnp.mean(xc * xc, axis=0, keepdims=True)
    h = xc * lax.rsqrt(var + 1e-5) * g3_ref[...] + bt3_ref[...]
    logits = jnp.dot(h, wm_ref[...], preferred_element_type=jnp.float32)
    logits = logits + bm_ref[...]
    m = jnp.max(logits, axis=1, keepdims=True)
    lse = m + jnp.log(jnp.sum(jnp.exp(logits - m), axis=1, keepdims=True))
    out_ref[...] = logits - lse


def _make_sc_kernels(perw):
    deg = pl.kernel(
        _deg_body,
        out_type=jax.ShapeDtypeStruct((NC, NPAD, 16), jnp.float32),
        mesh=_MESH,
        scratch_types=[
            pltpu.VMEM((perw, CH), jnp.int32),
            pltpu.VMEM((CH, 16), jnp.float32),
            pltpu.VMEM_SHARED((NPAD, 16), jnp.float32),
        ],
    )
    agg = pl.kernel(
        _agg_body,
        out_type=jax.ShapeDtypeStruct((NC, NPAD, D), jnp.float32),
        mesh=_MESH,
        scratch_types=[
            pltpu.VMEM((perw, CH), jnp.int32),
            pltpu.VMEM((perw, CH), jnp.int32),
            pltpu.VMEM((CH, D), jnp.float32),
            pltpu.VMEM_SHARED((NPAD, D), jnp.float32),
            pltpu.SemaphoreType.DMA,
        ],
    )
    return deg, agg


def kernel(x, edge_index, W1, b1, W2, b2, g1, bt1, g2, bt2, g3, bt3, Wm, bm):
    src = edge_index[0]
    dst = edge_index[1]
    E = src.shape[0]
    nchunks = -(-E // CH)
    perw = -(-nchunks // NW)
    perw = -(-perw // 8) * 8
    tot = perw * NW * CH
    pad = tot - E
    srcm = jnp.concatenate(
        [src, jnp.zeros((pad,), jnp.int32)]).reshape(perw * NW, CH)
    dstm = jnp.concatenate(
        [dst, jnp.full((pad,), N, jnp.int32)]).reshape(perw * NW, CH)
    ones = jnp.ones((CH, 16), jnp.float32)
    zeros16 = jnp.zeros((NPAD, 16), jnp.float32)
    zerosD = jnp.zeros((NPAD, D), jnp.float32)
    b1r, b2r = b1.reshape(1, D), b2.reshape(1, D)
    g1r, g2r, g3r = g1.reshape(1, D), g2.reshape(1, D), g3.reshape(1, D)
    bt1r, bt2r, bt3r = bt1.reshape(1, D), bt2.reshape(1, D), bt3.reshape(1, D)
    bmr = bm.reshape(1, DOUT)

    sc_deg, sc_agg = _make_sc_kernels(perw)

    degp = sc_deg(dstm, ones, zeros16)

    hs1, dinvp = pl.pallas_call(
        _tc1_body,
        out_shape=[
            jax.ShapeDtypeStruct((N, D), jnp.float32),
            jax.ShapeDtypeStruct((NPAD, 16), jnp.float32),
        ],
    )(x, W1, degp)

    a1 = sc_agg(srcm, dstm, hs1, zerosD)

    hs2 = pl.pallas_call(
        _tc2_body,
        out_shape=jax.ShapeDtypeStruct((N, D), jnp.float32),
    )(a1, hs1, dinvp, b1r, g1r, bt1r, W2)

    a2 = sc_agg(srcm, dstm, hs2, zerosD)

    out = pl.pallas_call(
        _tc3_body,
        out_shape=jax.ShapeDtypeStruct((N, DOUT), jnp.float32),
    )(a2, hs2, dinvp, b2r, g2r, bt2r, g3r, bt3r, Wm, bmr)

    return out


# speedup vs baseline: 26.4631x; 2.7872x over previous
import jax
import jax.numpy as jnp
from jax import lax
from jax.experimental import pallas as pl
from jax.experimental.pallas import tpu as pltpu
from jax.experimental.pallas import tpu_sc as plsc

N = 10000
D = 128
DOUT = 40
NPAD = 10240
CH = 128
NC = 2
NS = 16
NW = NC * NS
RPT = NPAD // NS

_MESH = plsc.VectorSubcoreMesh(
    core_axis_name="c", subcore_axis_name="s", num_cores=NC, num_subcores=NS)


def _deg_body(dstm, ones, zeros, out, didx_v, ones_v, acc_sh):
    c = lax.axis_index("c")
    s = lax.axis_index("s")
    w = s * NC + c
    perw = dstm.shape[0] // NW
    rows = pl.ds(s * RPT, RPT)
    pltpu.sync_copy(zeros.at[rows], acc_sh.at[rows])
    pltpu.sync_copy(ones, ones_v)
    pltpu.sync_copy(dstm.at[pl.ds(w * perw, perw)], didx_v)
    plsc.subcore_barrier()

    def step(j, carry):
        pltpu.sync_copy(ones_v, acc_sh.at[didx_v.at[j]], add=True)
        return carry

    lax.fori_loop(0, perw, step, 0)
    plsc.subcore_barrier()
    pltpu.sync_copy(acc_sh.at[rows], out.at[c, rows, :])


def _agg_body(srcm, dstm, hs, zeros, out, sidx_v, didx_v, rows0_v, rows1_v,
              acc_sh, gsem0, gsem1):
    c = lax.axis_index("c")
    s = lax.axis_index("s")
    w = s * NC + c
    perw = srcm.shape[0] // NW
    half = perw // 2
    rows = pl.ds(s * RPT, RPT)
    pltpu.sync_copy(zeros.at[rows], acc_sh.at[rows])
    plsc.subcore_barrier()

    def step(j2, carry):
        j = 2 * j2
        pltpu.make_async_copy(hs.at[sidx_v.at[j]], rows0_v, gsem0).wait()
        pltpu.async_copy(hs.at[sidx_v.at[j + 1]], rows1_v, gsem1)
        pltpu.sync_copy(rows0_v, acc_sh.at[didx_v.at[j]], add=True)
        jn = jnp.minimum(j + 2, half - 1)
        pltpu.make_async_copy(hs.at[sidx_v.at[j + 1]], rows1_v, gsem1).wait()
        pltpu.async_copy(hs.at[sidx_v.at[jn]], rows0_v, gsem0)
        pltpu.sync_copy(rows1_v, acc_sh.at[didx_v.at[j + 1]], add=True)
        return carry

    for h in range(2):
        base = w * perw + h * half
        pltpu.sync_copy(srcm.at[pl.ds(base, half)], sidx_v)
        pltpu.sync_copy(dstm.at[pl.ds(base, half)], didx_v)
        pltpu.async_copy(hs.at[sidx_v.at[0]], rows0_v, gsem0)
        lax.fori_loop(0, half // 2, step, 0)
        pltpu.make_async_copy(hs.at[sidx_v.at[half - 1]], rows0_v, gsem0).wait()
    plsc.subcore_barrier()
    pltpu.sync_copy(acc_sh.at[rows], out.at[c, rows, :])


def _tc1_body(x_ref, w1_ref, degp_ref, hs_ref, dinv_ref):
    deg = degp_ref[0] + degp_ref[1] + 1.0
    dinv = lax.rsqrt(deg)
    dinv_ref[...] = dinv
    h = jnp.dot(x_ref[...], w1_ref[...], preferred_element_type=jnp.float32)
    hs_ref[...] = h * dinv[:N, 0:1]


def _bn_relu(conv, g, bt):
    mu = jnp.mean(conv, axis=0, keepdims=True)
    xc = conv - mu
    var = jnp.mean(xc * xc, axis=0, keepdims=True)
    return jnp.maximum(xc * lax.rsqrt(var + 1e-5) * g + bt, 0.0)


def _tc2_body(aggp_ref, hs_ref, dinvp_ref, b1_ref, g1_ref, bt1_ref, w2_ref,
              hs2_ref):
    dcol = dinvp_ref[:, 0:1][:N]
    conv = (aggp_ref[0, :N, :] + aggp_ref[1, :N, :] + hs_ref[...]) * dcol
    h = _bn_relu(conv + b1_ref[...], g1_ref[...], bt1_ref[...])
    h2 = jnp.dot(h, w2_ref[...], preferred_element_type=jnp.float32)
    hs2_ref[...] = h2 * dcol


def _tc3_body(aggp_ref, hs2_ref, dinvp_ref, b2_ref, g2_ref, bt2_ref,
              g3_ref, bt3_ref, wm_ref, bm_ref, out_ref):
    dcol = dinvp_ref[:, 0:1][:N]
    conv = (aggp_ref[0, :N, :] + aggp_ref[1, :N, :] + hs2_ref[...]) * dcol
    h = _bn_relu(conv + b2_ref[...], g2_ref[...], bt2_ref[...])
    mu = jnp.mean(h, axis=0, keepdims=True)
    xc = h - mu
    var = jnp.mean(xc * xc, axis=0, keepdims=True)
    h = xc * lax.rsqrt(var + 1e-5) * g3_ref[...] + bt3_ref[...]
    logits = jnp.dot(h, wm_ref[...], preferred_element_type=jnp.float32)
    logits = logits + bm_ref[...]
    m = jnp.max(logits, axis=1, keepdims=True)
    lse = m + jnp.log(jnp.sum(jnp.exp(logits - m), axis=1, keepdims=True))
    out_ref[...] = logits - lse


def _make_sc_kernels(perw):
    deg = pl.kernel(
        _deg_body,
        out_type=jax.ShapeDtypeStruct((NC, NPAD, 16), jnp.float32),
        mesh=_MESH,
        scratch_types=[
            pltpu.VMEM((perw, CH), jnp.int32),
            pltpu.VMEM((CH, 16), jnp.float32),
            pltpu.VMEM_SHARED((NPAD, 16), jnp.float32),
        ],
    )
    agg = pl.kernel(
        _agg_body,
        out_type=jax.ShapeDtypeStruct((NC, NPAD, D), jnp.float32),
        mesh=_MESH,
        scratch_types=[
            pltpu.VMEM((perw // 2, CH), jnp.int32),
            pltpu.VMEM((perw // 2, CH), jnp.int32),
            pltpu.VMEM((CH, D), jnp.float32),
            pltpu.VMEM((CH, D), jnp.float32),
            pltpu.VMEM_SHARED((NPAD, D), jnp.float32),
            pltpu.SemaphoreType.DMA,
            pltpu.SemaphoreType.DMA,
        ],
    )
    return deg, agg


def kernel(x, edge_index, W1, b1, W2, b2, g1, bt1, g2, bt2, g3, bt3, Wm, bm):
    src = edge_index[0]
    dst = edge_index[1]
    E = src.shape[0]
    nchunks = -(-E // CH)
    perw = -(-nchunks // NW)
    perw = -(-perw // 8) * 8
    tot = perw * NW * CH
    pad = tot - E
    ar = lax.iota(jnp.int32, pad)
    srcm = jnp.concatenate(
        [src, ar % N]).reshape(perw * NW, CH)
    dstm = jnp.concatenate(
        [dst, N + ar % (NPAD - N)]).reshape(perw * NW, CH)
    ones = jnp.ones((CH, 16), jnp.float32)
    zeros16 = jnp.zeros((NPAD, 16), jnp.float32)
    zerosD = jnp.zeros((NPAD, D), jnp.float32)
    b1r, b2r = b1.reshape(1, D), b2.reshape(1, D)
    g1r, g2r, g3r = g1.reshape(1, D), g2.reshape(1, D), g3.reshape(1, D)
    bt1r, bt2r, bt3r = bt1.reshape(1, D), bt2.reshape(1, D), bt3.reshape(1, D)
    bmr = bm.reshape(1, DOUT)

    sc_deg, sc_agg = _make_sc_kernels(perw)

    degp = sc_deg(dstm, ones, zeros16)

    hs1, dinvp = pl.pallas_call(
        _tc1_body,
        out_shape=[
            jax.ShapeDtypeStruct((N, D), jnp.float32),
            jax.ShapeDtypeStruct((NPAD, 16), jnp.float32),
        ],
    )(x, W1, degp)

    a1 = sc_agg(srcm, dstm, hs1, zerosD)

    hs2 = pl.pallas_call(
        _tc2_body,
        out_shape=jax.ShapeDtypeStruct((N, D), jnp.float32),
    )(a1, hs1, dinvp, b1r, g1r, bt1r, W2)

    a2 = sc_agg(srcm, dstm, hs2, zerosD)

    out = pl.pallas_call(
        _tc3_body,
        out_shape=jax.ShapeDtypeStruct((N, DOUT), jnp.float32),
    )(a2, hs2, dinvp, b2r, g2r, bt2r, g3r, bt3r, Wm, bmr)

    return out
